# baseline (speedup 1.0000x reference)
import jax
import jax.numpy as jnp
from jax.experimental import pallas as pl

N_VARS = 5
TOTAL_MEM = 24
NUM_OC = 18
N_OUT = 2
BB = 1024


def _anfis_block(x_ref, p_ref, ir_ref, out_ref):
    rpad = ir_ref.shape[1] // N_VARS
    bb = x_ref.shape[0]
    c_t = jnp.broadcast_to(p_ref[:TOTAL_MEM, 0:1], (TOTAL_MEM, 128))
    s_t = jnp.broadcast_to(p_ref[:TOTAL_MEM, 1:2], (TOTAL_MEM, 128))
    vm_t = jnp.broadcast_to(p_ref[:TOTAL_MEM, 2:3], (TOTAL_MEM, 128))
    inv_t = 0.5 / (s_t ** 2)

    parts = []
    for ch in range(bb // 128):
        xc = jnp.swapaxes(x_ref[ch * 128:(ch + 1) * 128, :], 0, 1)
        xv = jnp.zeros((TOTAL_MEM, 128), jnp.float32)
        for v in range(N_VARS):
            xv = jnp.where(vm_t == float(v),
                           jnp.broadcast_to(xc[v:v + 1, :], (TOTAL_MEM, 128)),
                           xv)
        f = jnp.exp(-((xv - c_t) ** 2) * inv_t)
        fhi = f.astype(jnp.bfloat16)
        flo = (f - fhi.astype(jnp.float32)).astype(jnp.bfloat16)
        parts.append(jnp.concatenate([fhi, flo], axis=0))
    lhs_t = jnp.concatenate(parts, axis=1)

    idx = ir_ref[0, :]
    oh = (jax.lax.broadcasted_iota(jnp.int32, (TOTAL_MEM, N_VARS * rpad), 0)
          == idx[None, :]).astype(jnp.bfloat16)
    oh2 = jnp.concatenate([oh, oh], axis=0)
    G = jax.lax.dot_general(lhs_t, oh2, (((0,), (0,)), ((), ())),
                            preferred_element_type=jnp.float32)

    oc = p_ref[:NUM_OC, 3]
    ows = []
    for j in range(N_OUT):
        orj = ir_ref[1 + j, :rpad]
        owj = jnp.zeros((1, rpad), jnp.float32)
        for k in range(NUM_OC):
            owj = jnp.where((orj == k)[None, :], oc[k], owj)
        ows.append(owj)

    a0 = jnp.zeros((bb, 128), jnp.float32)
    a1 = jnp.zeros((bb, 128), jnp.float32)
    at = jnp.zeros((bb, 128), jnp.float32)
    for kk in range(rpad // 128):
        gbase = kk * N_VARS * 128
        m = G[:, gbase:gbase + 128]
        for v in range(1, N_VARS):
            m = jnp.minimum(m, G[:, gbase + v * 128:gbase + (v + 1) * 128])
        base = kk * 128
        a0 = a0 + m * ows[0][:, base:base + 128]
        a1 = a1 + m * ows[1][:, base:base + 128]
        at = at + m
    acc0 = jnp.sum(a0, axis=1, keepdims=True)
    acc1 = jnp.sum(a1, axis=1, keepdims=True)
    total = jnp.sum(at, axis=1, keepdims=True)
    acc = jnp.concatenate([acc0, acc1], axis=1)
    res = jnp.tanh(acc / jnp.maximum(total, 1e-12))
    scale = p_ref[:N_OUT, 4]
    bias = p_ref[:N_OUT, 5]
    out_ref[:, :] = res * scale[None, :] + bias[None, :]


def kernel(x, centers, sigmas, out_centers, output_scaling, output_bias,
           input_rules, output_rules, var_of_mem):
    b, nv = x.shape
    r = input_rules.shape[0]
    rpad = ((r + 127) // 128) * 128
    ri = jnp.full((N_VARS, rpad), TOTAL_MEM + 7, jnp.int32)
    ri = ri.at[:, :r].set(input_rules.T)
    ri = ri.reshape(N_VARS, rpad // 128, 128).swapaxes(0, 1)
    ir = jnp.full((8, N_VARS * rpad), TOTAL_MEM + 7, jnp.int32)
    ir = ir.at[0:1, :].set(ri.reshape(1, N_VARS * rpad))
    ir = ir.at[1:1 + N_OUT, :r].set(output_rules.T)
    p = jnp.zeros((128, 8), jnp.float32)
    p = p.at[:TOTAL_MEM, 0].set(centers)
    p = p.at[:TOTAL_MEM, 1].set(sigmas)
    p = p.at[:TOTAL_MEM, 2].set(var_of_mem.astype(jnp.float32))
    p = p.at[:NUM_OC, 3].set(out_centers)
    p = p.at[:N_OUT, 4].set(output_scaling)
    p = p.at[:N_OUT, 5].set(output_bias)

    full = lambda shape: pl.BlockSpec(shape, lambda i: (0, 0))
    out = pl.pallas_call(
        _anfis_block,
        grid=(b // BB,),
        in_specs=[
            pl.BlockSpec((BB, nv), lambda i: (i, 0)),
            full((128, 8)),
            full((8, N_VARS * rpad)),
        ],
        out_specs=pl.BlockSpec((BB, N_OUT), lambda i: (i, 0)),
        out_shape=jax.ShapeDtypeStruct((b, N_OUT), jnp.float32),
    )(x, p, ir)
    return out


# speedup vs baseline: 1.6987x; 1.6987x over previous
import jax
import jax.numpy as jnp
from jax.experimental import pallas as pl

N_VARS = 5
TOTAL_MEM = 24
NUM_OC = 18
N_OUT = 2
BB = 1024


def _anfis_block(xt_ref, c_ref, s_ref, oc_ref, scale_ref, bias_ref,
                 ir_ref, vm_ref, out_ref):
    rpad = ir_ref.shape[1] // N_VARS
    bb = xt_ref.shape[1]
    c_t = c_ref[:, :]
    inv_t = 0.5 / (s_ref[:, :] ** 2)
    vm_t = vm_ref[:, :]

    parts = []
    for ch in range(bb // 128):
        xc = xt_ref[:, ch * 128:(ch + 1) * 128]
        xv = jnp.zeros((TOTAL_MEM, 128), jnp.float32)
        for v in range(N_VARS):
            xv = jnp.where(vm_t == v,
                           jnp.broadcast_to(xc[v:v + 1, :], (TOTAL_MEM, 128)),
                           xv)
        f = jnp.exp(-((xv - c_t) ** 2) * inv_t)
        fhi = f.astype(jnp.bfloat16)
        flo = (f - fhi.astype(jnp.float32)).astype(jnp.bfloat16)
        parts.append(jnp.concatenate([fhi, flo], axis=0))
    lhs_t = jnp.concatenate(parts, axis=1)

    idx = ir_ref[0, :]
    oh = (jax.lax.broadcasted_iota(jnp.int32, (TOTAL_MEM, N_VARS * rpad), 0)
          == idx[None, :]).astype(jnp.bfloat16)
    oh2 = jnp.concatenate([oh, oh], axis=0)
    G = jax.lax.dot_general(lhs_t, oh2, (((0,), (0,)), ((), ())),
                            preferred_element_type=jnp.float32)

    oc = oc_ref[0, :]
    ows = []
    for j in range(N_OUT):
        orj = ir_ref[1 + j, :rpad]
        owj = jnp.zeros((1, rpad), jnp.float32)
        for k in range(NUM_OC):
            owj = jnp.where((orj == k)[None, :], oc[k], owj)
        ows.append(owj)

    a0 = jnp.zeros((bb, 128), jnp.float32)
    a1 = jnp.zeros((bb, 128), jnp.float32)
    at = jnp.zeros((bb, 128), jnp.float32)
    for kk in range(rpad // 128):
        gbase = kk * N_VARS * 128
        m = G[:, gbase:gbase + 128]
        for v in range(1, N_VARS):
            m = jnp.minimum(m, G[:, gbase + v * 128:gbase + (v + 1) * 128])
        base = kk * 128
        a0 = a0 + m * ows[0][:, base:base + 128]
        a1 = a1 + m * ows[1][:, base:base + 128]
        at = at + m
    acc0 = jnp.sum(a0, axis=1, keepdims=True)
    acc1 = jnp.sum(a1, axis=1, keepdims=True)
    total = jnp.sum(at, axis=1, keepdims=True)
    acc = jnp.concatenate([acc0, acc1], axis=1)
    res = jnp.tanh(acc / jnp.maximum(total, 1e-12))
    out_ref[:, :] = res * scale_ref[0, :][None, :] + bias_ref[0, :][None, :]


def kernel(x, centers, sigmas, out_centers, output_scaling, output_bias,
           input_rules, output_rules, var_of_mem):
    b, nv = x.shape
    r = input_rules.shape[0]
    rpad = ((r + 127) // 128) * 128
    xt = jnp.zeros((8, b), jnp.float32).at[:nv, :].set(x.T)
    ri = jnp.full((N_VARS, rpad), TOTAL_MEM + 7, jnp.int32)
    ri = ri.at[:, :r].set(input_rules.T)
    ri = ri.reshape(N_VARS, rpad // 128, 128).swapaxes(0, 1)
    ir = jnp.full((8, N_VARS * rpad), TOTAL_MEM + 7, jnp.int32)
    ir = ir.at[0:1, :].set(ri.reshape(1, N_VARS * rpad))
    ir = ir.at[1:1 + N_OUT, :r].set(output_rules.T)
    c2 = jnp.broadcast_to(centers[:, None], (TOTAL_MEM, 128))
    s2 = jnp.broadcast_to(sigmas[:, None], (TOTAL_MEM, 128))
    vm2 = jnp.broadcast_to(var_of_mem[:, None], (TOTAL_MEM, 128))
    oc2 = out_centers.reshape(1, -1)
    sc2 = output_scaling.reshape(1, N_OUT)
    bi2 = output_bias.reshape(1, N_OUT)

    full = lambda shape: pl.BlockSpec(shape, lambda i: (0, 0))
    out = pl.pallas_call(
        _anfis_block,
        grid=(b // BB,),
        in_specs=[
            pl.BlockSpec((8, BB), lambda i: (0, i)),
            full((TOTAL_MEM, 128)),
            full((TOTAL_MEM, 128)),
            full((1, oc2.shape[1])),
            full((1, N_OUT)),
            full((1, N_OUT)),
            full((8, N_VARS * rpad)),
            full((TOTAL_MEM, 128)),
        ],
        out_specs=pl.BlockSpec((BB, N_OUT), lambda i: (i, 0)),
        out_shape=jax.ShapeDtypeStruct((b, N_OUT), jnp.float32),
    )(xt, c2, s2, oc2, sc2, bi2, ir, vm2)
    return out


# speedup vs baseline: 1.8980x; 1.1173x over previous
import jax
import jax.numpy as jnp
from jax.experimental import pallas as pl

N_VARS = 5
TOTAL_MEM = 24
NUM_OC = 18
N_OUT = 2
BB = 1024


def _anfis_block(xt_ref, p_ref, ir_ref, out_ref):
    rpad = ir_ref.shape[1]
    bb = xt_ref.shape[1]
    c_t = jnp.broadcast_to(
        jnp.swapaxes(p_ref[0:1, :TOTAL_MEM], 0, 1), (TOTAL_MEM, 128))
    s_t = jnp.broadcast_to(
        jnp.swapaxes(p_ref[1:2, :TOTAL_MEM], 0, 1), (TOTAL_MEM, 128))
    vm_t = jnp.broadcast_to(
        jnp.swapaxes(p_ref[2:3, :TOTAL_MEM], 0, 1), (TOTAL_MEM, 128))
    inv_t = 0.5 / (s_t ** 2)

    parts = []
    for ch in range(bb // 128):
        xc = xt_ref[:, ch * 128:(ch + 1) * 128]
        xv = jnp.zeros((TOTAL_MEM, 128), jnp.float32)
        for v in range(N_VARS):
            xv = jnp.where(vm_t == float(v),
                           jnp.broadcast_to(xc[v:v + 1, :], (TOTAL_MEM, 128)),
                           xv)
        f = jnp.exp(-((xv - c_t) ** 2) * inv_t)
        fhi = f.astype(jnp.bfloat16)
        flo = (f - fhi.astype(jnp.float32)).astype(jnp.bfloat16)
        parts.append(jnp.concatenate([fhi, flo], axis=0))
    lhs_t = jnp.concatenate(parts, axis=1)

    io = jax.lax.broadcasted_iota(jnp.int32, (TOTAL_MEM, 128), 0)
    cols = []
    for kk in range(rpad // 128):
        for v in range(N_VARS):
            idx = ir_ref[v, kk * 128:(kk + 1) * 128]
            cols.append(io == idx[None, :])
    ohz = jnp.concatenate(cols, axis=1).astype(jnp.bfloat16)
    oh2 = jnp.concatenate([ohz, ohz], axis=0)
    G = jax.lax.dot_general(lhs_t, oh2, (((0,), (0,)), ((), ())),
                            preferred_element_type=jnp.float32)

    oc = p_ref[3, :NUM_OC]
    ows = []
    for j in range(N_OUT):
        orj = ir_ref[N_VARS + j, :]
        owj = jnp.zeros((1, rpad), jnp.float32)
        for k in range(NUM_OC):
            owj = jnp.where((orj == k)[None, :], oc[k], owj)
        ows.append(owj)

    a0 = jnp.zeros((bb, 128), jnp.float32)
    a1 = jnp.zeros((bb, 128), jnp.float32)
    at = jnp.zeros((bb, 128), jnp.float32)
    for kk in range(rpad // 128):
        gbase = kk * N_VARS * 128
        m = G[:, gbase:gbase + 128]
        for v in range(1, N_VARS):
            m = jnp.minimum(m, G[:, gbase + v * 128:gbase + (v + 1) * 128])
        base = kk * 128
        a0 = a0 + m * ows[0][:, base:base + 128]
        a1 = a1 + m * ows[1][:, base:base + 128]
        at = at + m
    acc0 = jnp.sum(a0, axis=1, keepdims=True)
    acc1 = jnp.sum(a1, axis=1, keepdims=True)
    total = jnp.sum(at, axis=1, keepdims=True)
    acc = jnp.concatenate([acc0, acc1], axis=1)
    res = jnp.tanh(acc / jnp.maximum(total, 1e-12))
    out_ref[:, :] = (res * p_ref[4:5, :N_OUT] + p_ref[5:6, :N_OUT])


def kernel(x, centers, sigmas, out_centers, output_scaling, output_bias,
           input_rules, output_rules, var_of_mem):
    b, nv = x.shape
    r = input_rules.shape[0]
    rpad = ((r + 127) // 128) * 128
    xt = jnp.concatenate([x.T, jnp.zeros((8 - nv, b), jnp.float32)], axis=0)
    ir = jnp.concatenate([
        jnp.pad(input_rules.T, ((0, 0), (0, rpad - r)),
                constant_values=TOTAL_MEM + 7),
        jnp.pad(output_rules.T, ((0, 0), (0, rpad - r)),
                constant_values=NUM_OC + 7),
        jnp.full((1, rpad), TOTAL_MEM + 7, jnp.int32),
    ], axis=0)
    pad128 = lambda v: jnp.pad(v.astype(jnp.float32), (0, 128 - v.shape[0]))
    p = jnp.stack([
        pad128(centers), pad128(sigmas), pad128(var_of_mem),
        pad128(out_centers), pad128(output_scaling), pad128(output_bias),
        jnp.zeros(128, jnp.float32), jnp.zeros(128, jnp.float32),
    ], axis=0)

    full = lambda shape: pl.BlockSpec(shape, lambda i: (0, 0))
    out = pl.pallas_call(
        _anfis_block,
        grid=(b // BB,),
        in_specs=[
            pl.BlockSpec((8, BB), lambda i: (0, i)),
            full((8, 128)),
            full((8, rpad)),
        ],
        out_specs=pl.BlockSpec((BB, N_OUT), lambda i: (i, 0)),
        out_shape=jax.ShapeDtypeStruct((b, N_OUT), jnp.float32),
    )(xt, p, ir)
    return out
